# baseline (speedup 1.0000x reference)
import functools

import jax
import jax.numpy as jnp
from jax import lax
from jax.experimental import pallas as pl
from jax.experimental.pallas import tpu as pltpu
from jax.experimental.pallas import tpu_sc as plsc

NC = 2
NS = 16
NW = NC * NS


@functools.lru_cache(maxsize=None)
def _make_gather(batch: int, vocab: int, dim: int, k: int, ring: int):
    bpw = batch // NW
    nchunk = bpw // k
    rounds = nchunk // ring
    rem = nchunk - rounds * ring
    mesh = plsc.VectorSubcoreMesh(
        core_axis_name="c", subcore_axis_name="s",
        num_cores=NC, num_subcores=NS,
    )

    @functools.partial(
        pl.kernel,
        out_type=jax.ShapeDtypeStruct((batch, dim), jnp.float32),
        mesh=mesh,
        scratch_types=[
            pltpu.VMEM((nchunk, k), jnp.int32),
            [pltpu.VMEM((k, dim), jnp.float32)] * ring,
            [pltpu.SemaphoreType.DMA] * ring,
            [pltpu.SemaphoreType.DMA] * ring,
        ],
    )
    def gather_kernel(idx_hbm, table_hbm, out_hbm, idx_v, bufs, gsems, ssems):
        wid = lax.axis_index("s") * NC + lax.axis_index("c")
        base = wid * bpw
        pltpu.sync_copy(idx_hbm.at[wid], idx_v)

        def start_g(j, t):
            pltpu.async_copy(table_hbm.at[idx_v.at[j]], bufs[t], gsems[t])

        def wait_g(t):
            pltpu.make_async_copy(
                table_hbm.at[idx_v.at[0]], bufs[t], gsems[t]).wait()

        def start_s(j, t):
            pltpu.async_copy(
                bufs[t], out_hbm.at[pl.ds(base + j * k, k)], ssems[t])

        def wait_s(t):
            pltpu.make_async_copy(
                bufs[t], out_hbm.at[pl.ds(base, k)], ssems[t]).wait()

        for t in range(ring):
            start_g(t, t)

        @pl.loop(0, rounds)
        def _round(i):
            j = i * ring
            for t in range(ring):
                wait_g(t)
                start_s(j + t, t)
            for t in range(ring):
                @pl.when(j + ring + t < nchunk)
                def _refill():
                    wait_s(t)
                    start_g(j + ring + t, t)

        for t in range(rem):
            wait_g(t)
            start_s(rounds * ring + t, t)
        for t in range(ring):
            wait_s(t)

    return gather_kernel


def kernel(idx, embedding):
    if idx.ndim == 2:
        idx = jnp.squeeze(idx, axis=-1)
    batch = idx.shape[0]
    vocab, dim = embedding.shape
    k, ring = 4, 3
    idx3 = idx.astype(jnp.int32).reshape(NW, (batch // NW) // k, k)
    return _make_gather(batch, vocab, dim, k, ring)(idx3, embedding)


# speedup vs baseline: 1.0197x; 1.0197x over previous
import functools

import jax
import jax.numpy as jnp
from jax import lax
from jax.experimental import pallas as pl
from jax.experimental.pallas import tpu as pltpu
from jax.experimental.pallas import tpu_sc as plsc

NC = 2
NS = 16
NW = NC * NS


@functools.lru_cache(maxsize=None)
def _make_gather(batch: int, vocab: int, dim: int, k: int, ring: int):
    bpw = batch // NW
    nchunk = bpw // k
    rounds = nchunk // ring
    rem = nchunk - rounds * ring
    mesh = plsc.VectorSubcoreMesh(
        core_axis_name="c", subcore_axis_name="s",
        num_cores=NC, num_subcores=NS,
    )

    @functools.partial(
        pl.kernel,
        out_type=jax.ShapeDtypeStruct((batch, dim), jnp.float32),
        mesh=mesh,
        scratch_types=[
            pltpu.VMEM((nchunk, k), jnp.int32),
            [pltpu.VMEM((k, dim), jnp.float32)] * ring,
            [pltpu.SemaphoreType.DMA] * ring,
            [pltpu.SemaphoreType.DMA] * ring,
        ],
    )
    def gather_kernel(idx_hbm, table_hbm, out_hbm, idx_v, bufs, gsems, ssems):
        wid = lax.axis_index("s") * NC + lax.axis_index("c")
        base = wid * bpw
        pltpu.sync_copy(idx_hbm.at[wid], idx_v)

        def start_g(j, t):
            pltpu.async_copy(table_hbm.at[idx_v.at[j]], bufs[t], gsems[t])

        def wait_g(t):
            pltpu.make_async_copy(
                table_hbm.at[idx_v.at[0]], bufs[t], gsems[t]).wait()

        def start_s(j, t):
            pltpu.async_copy(
                bufs[t], out_hbm.at[pl.ds(base + j * k, k)], ssems[t])

        def wait_s(t):
            pltpu.make_async_copy(
                bufs[t], out_hbm.at[pl.ds(base, k)], ssems[t]).wait()

        for t in range(ring):
            start_g(t, t)

        @pl.loop(0, rounds)
        def _round(i):
            j = i * ring
            for t in range(ring):
                wait_g(t)
                start_s(j + t, t)
            for t in range(ring):
                @pl.when(j + ring + t < nchunk)
                def _refill():
                    wait_s(t)
                    start_g(j + ring + t, t)

        for t in range(rem):
            wait_g(t)
            start_s(rounds * ring + t, t)
        for t in range(ring):
            wait_s(t)

    return gather_kernel


def kernel(idx, embedding):
    if idx.ndim == 2:
        idx = jnp.squeeze(idx, axis=-1)
    batch = idx.shape[0]
    vocab, dim = embedding.shape
    k, ring = 2, 6
    idx3 = idx.astype(jnp.int32).reshape(NW, (batch // NW) // k, k)
    return _make_gather(batch, vocab, dim, k, ring)(idx3, embedding)


# speedup vs baseline: 1.0227x; 1.0030x over previous
import functools

import jax
import jax.numpy as jnp
from jax import lax
from jax.experimental import pallas as pl
from jax.experimental.pallas import tpu as pltpu
from jax.experimental.pallas import tpu_sc as plsc

NC = 2
NS = 16
NW = NC * NS


@functools.lru_cache(maxsize=None)
def _make_gather(batch: int, vocab: int, dim: int, k: int, ring: int):
    bpw = batch // NW
    nchunk = bpw // k
    rounds = nchunk // ring
    rem = nchunk - rounds * ring
    mesh = plsc.VectorSubcoreMesh(
        core_axis_name="c", subcore_axis_name="s",
        num_cores=NC, num_subcores=NS,
    )

    @functools.partial(
        pl.kernel,
        out_type=jax.ShapeDtypeStruct((batch, dim), jnp.float32),
        mesh=mesh,
        scratch_types=[
            pltpu.VMEM((nchunk, k), jnp.int32),
            [pltpu.VMEM((k, dim), jnp.float32)] * ring,
            [pltpu.SemaphoreType.DMA] * ring,
            [pltpu.SemaphoreType.DMA] * ring,
        ],
    )
    def gather_kernel(idx_hbm, table_hbm, out_hbm, idx_v, bufs, gsems, ssems):
        wid = lax.axis_index("s") * NC + lax.axis_index("c")
        base = wid * bpw
        pltpu.sync_copy(idx_hbm.at[wid], idx_v)

        def start_g(j, t):
            pltpu.async_copy(table_hbm.at[idx_v.at[j]], bufs[t], gsems[t])

        def wait_g(t):
            pltpu.make_async_copy(
                table_hbm.at[idx_v.at[0]], bufs[t], gsems[t]).wait()

        def start_s(j, t):
            pltpu.async_copy(
                bufs[t], out_hbm.at[pl.ds(base + j * k, k)], ssems[t])

        def wait_s(t):
            pltpu.make_async_copy(
                bufs[t], out_hbm.at[pl.ds(base, k)], ssems[t]).wait()

        for t in range(ring):
            start_g(t, t)

        @pl.loop(0, rounds)
        def _round(i):
            j = i * ring
            for t in range(ring):
                wait_g(t)
                start_s(j + t, t)
            for t in range(ring):
                @pl.when(j + ring + t < nchunk)
                def _refill():
                    wait_s(t)
                    start_g(j + ring + t, t)

        for t in range(rem):
            wait_g(t)
            start_s(rounds * ring + t, t)
        for t in range(ring):
            wait_s(t)

    return gather_kernel


def kernel(idx, embedding):
    if idx.ndim == 2:
        idx = jnp.squeeze(idx, axis=-1)
    batch = idx.shape[0]
    vocab, dim = embedding.shape
    k, ring = 2, 4
    idx3 = idx.astype(jnp.int32).reshape(NW, (batch // NW) // k, k)
    return _make_gather(batch, vocab, dim, k, ring)(idx3, embedding)


# speedup vs baseline: 1.0235x; 1.0008x over previous
import functools

import jax
import jax.numpy as jnp
from jax import lax
from jax.experimental import pallas as pl
from jax.experimental.pallas import tpu as pltpu
from jax.experimental.pallas import tpu_sc as plsc

NC = 2
NS = 16
NW = NC * NS


@functools.lru_cache(maxsize=None)
def _make_gather(batch: int, vocab: int, dim: int, k: int, ring: int):
    bpw = batch // NW
    nchunk = bpw // k
    rounds = nchunk // ring
    rem = nchunk - rounds * ring
    mesh = plsc.VectorSubcoreMesh(
        core_axis_name="c", subcore_axis_name="s",
        num_cores=NC, num_subcores=NS,
    )

    @functools.partial(
        pl.kernel,
        out_type=jax.ShapeDtypeStruct((batch, dim), jnp.float32),
        mesh=mesh,
        compiler_params=pltpu.CompilerParams(use_tc_tiling_on_sc=True),
        scratch_types=[
            pltpu.VMEM((nchunk, k), jnp.int32),
            [pltpu.VMEM((k, dim), jnp.float32)] * ring,
            [pltpu.SemaphoreType.DMA] * ring,
            [pltpu.SemaphoreType.DMA] * ring,
        ],
    )
    def gather_kernel(idx_hbm, table_hbm, out_hbm, idx_v, bufs, gsems, ssems):
        wid = lax.axis_index("s") * NC + lax.axis_index("c")
        base = wid * bpw
        pltpu.sync_copy(idx_hbm.at[wid], idx_v)

        def start_g(j, t):
            pltpu.async_copy(table_hbm.at[idx_v.at[j]], bufs[t], gsems[t])

        def wait_g(t):
            pltpu.make_async_copy(
                table_hbm.at[idx_v.at[0]], bufs[t], gsems[t]).wait()

        def start_s(j, t):
            pltpu.async_copy(
                bufs[t], out_hbm.at[pl.ds(base + j * k, k)], ssems[t])

        def wait_s(t):
            pltpu.make_async_copy(
                bufs[t], out_hbm.at[pl.ds(base, k)], ssems[t]).wait()

        for t in range(ring):
            start_g(t, t)

        @pl.loop(0, rounds)
        def _round(i):
            j = i * ring
            for t in range(ring):
                wait_g(t)
                start_s(j + t, t)
            for t in range(ring):
                @pl.when(j + ring + t < nchunk)
                def _refill():
                    wait_s(t)
                    start_g(j + ring + t, t)

        for t in range(rem):
            wait_g(t)
            start_s(rounds * ring + t, t)
        for t in range(ring):
            wait_s(t)

    return gather_kernel


def kernel(idx, embedding):
    if idx.ndim == 2:
        idx = jnp.squeeze(idx, axis=-1)
    batch = idx.shape[0]
    vocab, dim = embedding.shape
    k, ring = 2, 4
    idx3 = idx.astype(jnp.int32).reshape(NW, (batch // NW) // k, k)
    return _make_gather(batch, vocab, dim, k, ring)(idx3, embedding)
